# baseline (speedup 1.0000x reference)
import jax
import jax.numpy as jnp
from jax import lax
from jax.experimental import pallas as pl
from jax.experimental.pallas import tpu as pltpu
from jax.experimental.pallas import tpu_sc as plsc

N, D = 10000, 256
E = 160000
NUM_REL = 500

NC, NS, L = 2, 16, 16
NW = NC * NS
PW = E // NW
CH = 96
LAST_START = PW - CH
NCHUNK = 54
NVEC2 = D // (2 * L)


def _body(enc_hbm, h_hbm, r_hbm, t_hbm, rel_hbm, out_hbm,
          idxh_v, idxr_v, idxt_v, eh_v, et_v, rel_v, outv_v, sems):
    wid = lax.axis_index("s") * NC + lax.axis_index("c")
    base = wid * PW
    lane = lax.iota(jnp.int32, L)

    pltpu.sync_copy(rel_hbm, rel_v)

    def chunk_start(c):
        return base + jnp.minimum(c * CH, LAST_START)

    def fetch(c, b):
        start = chunk_start(c)
        pltpu.sync_copy(h_hbm.at[pl.ds(start, CH)], idxh_v.at[b])
        pltpu.sync_copy(r_hbm.at[pl.ds(start, CH)], idxr_v.at[b])
        pltpu.sync_copy(t_hbm.at[pl.ds(start, CH)], idxt_v.at[b])
        pltpu.async_copy(enc_hbm.at[idxh_v.at[b]], eh_v.at[b], sems.at[b])
        pltpu.async_copy(enc_hbm.at[idxt_v.at[b]], et_v.at[b], sems.at[b])

    def drain(b):
        pltpu.make_async_copy(enc_hbm.at[pl.ds(0, CH)], eh_v.at[b],
                              sems.at[b]).wait()
        pltpu.make_async_copy(enc_hbm.at[pl.ds(0, CH)], et_v.at[b],
                              sems.at[b]).wait()

    def compute(c, b):
        def row_sum(row, ridx):
            acc = [jnp.zeros((L,), jnp.float32) for _ in range(4)]
            for i in range(NVEC2):
                sl = pl.ds(i * L, L)
                p = (plsc.bitcast(eh_v[b, row, sl], jnp.bfloat16)
                     * plsc.bitcast(rel_v[ridx, sl], jnp.bfloat16)
                     * plsc.bitcast(et_v[b, row, sl], jnp.bfloat16))
                pa, pb = plsc.unpack(p, format=plsc.PackFormat.INTERLEAVED)
                k = 2 * (i % 2)
                acc[k] = acc[k] + pa
                acc[k + 1] = acc[k + 1] + pb
            return jnp.sum((acc[0] + acc[1]) + (acc[2] + acc[3]))

        def g_body(g, _):
            rvec = idxr_v[b, pl.ds(g * L, L)]
            parts = []
            for j in range(L):
                s = row_sum(g * L + j, rvec[j])
                parts.append(jnp.where(lane == j, s, 0.0))
            while len(parts) > 1:
                parts = [parts[m] + parts[m + 1]
                         for m in range(0, len(parts), 2)]
            outv_v[pl.ds(g * L, L)] = parts[0]
            return _

        lax.fori_loop(0, CH // L, g_body, None, unroll=False)
        pltpu.sync_copy(outv_v, out_hbm.at[pl.ds(chunk_start(c), CH)])

    fetch(0, 0)

    def outer(i, _):
        for b in (0, 1):
            c = i * 2 + b

            @pl.when(c < NCHUNK - 1)
            def _fire():
                fetch(c + 1, 1 - b)

            drain(b)
            compute(c, b)
        return _

    lax.fori_loop(0, NCHUNK // 2, outer, None, unroll=False)


@jax.jit
def _dist_mult_sc(enc, h, r, t, rel_weight):
    mesh = plsc.VectorSubcoreMesh(core_axis_name="c", subcore_axis_name="s",
                                  num_cores=NC, num_subcores=NS)
    return pl.kernel(
        _body,
        out_type=jax.ShapeDtypeStruct((E,), jnp.float32),
        mesh=mesh,
        compiler_params=pltpu.CompilerParams(needs_layout_passes=False),
        scratch_types=[
            pltpu.VMEM((2, CH), jnp.int32),
            pltpu.VMEM((2, CH), jnp.int32),
            pltpu.VMEM((2, CH), jnp.int32),
            pltpu.VMEM((2, CH, D // 2), jnp.int32),
            pltpu.VMEM((2, CH, D // 2), jnp.int32),
            pltpu.VMEM((NUM_REL, D // 2), jnp.int32),
            pltpu.VMEM((CH,), jnp.float32),
            pltpu.SemaphoreType.DMA((2,)),
        ],
    )(enc, h, r, t, rel_weight)


def _pack_i32(table):
    tb = table.astype(jnp.bfloat16)
    return lax.bitcast_convert_type(tb.reshape(table.shape[0], -1, 2),
                                    jnp.int32)


def kernel(enc, h, r, t, rel_weight):
    return _dist_mult_sc(_pack_i32(enc),
                         jnp.asarray(h, jnp.int32),
                         jnp.asarray(r, jnp.int32),
                         jnp.asarray(t, jnp.int32),
                         _pack_i32(rel_weight))


# speedup vs baseline: 1.1918x; 1.1918x over previous
import jax
import jax.numpy as jnp
from jax import lax
from jax.experimental import pallas as pl
from jax.experimental.pallas import tpu as pltpu
from jax.experimental.pallas import tpu_sc as plsc

N, D = 10000, 256
E = 160000
NUM_REL = 500

NC, NS, L = 2, 16, 16
NW = NC * NS
PW = E // NW
CH = 160
LAST_START = PW - CH
NCHUNK = 32
NVEC2 = D // (2 * L)


def _body(enc_hbm, h_hbm, r_hbm, t_hbm, rel_hbm, out_hbm,
          idxh_v, idxr_v, idxt_v, eh_v, rr_v, et_v, outv_v, sems):
    wid = lax.axis_index("s") * NC + lax.axis_index("c")
    base = wid * PW
    lane = lax.iota(jnp.int32, L)

    def chunk_start(c):
        return base + jnp.minimum(c * CH, LAST_START)

    def fetch(c, b):
        start = chunk_start(c)
        ib = pl.ds(b * CH, CH)
        pltpu.sync_copy(h_hbm.at[pl.ds(start, CH)], idxh_v.at[ib])
        pltpu.sync_copy(r_hbm.at[pl.ds(start, CH)], idxr_v.at[ib])
        pltpu.sync_copy(t_hbm.at[pl.ds(start, CH)], idxt_v.at[ib])
        pltpu.async_copy(enc_hbm.at[idxh_v.at[ib]], eh_v.at[b], sems.at[b])
        pltpu.async_copy(rel_hbm.at[idxr_v.at[ib]], rr_v.at[b], sems.at[b])
        pltpu.async_copy(enc_hbm.at[idxt_v.at[ib]], et_v.at[b], sems.at[b])

    def drain(b):
        pltpu.make_async_copy(enc_hbm.at[pl.ds(0, CH)], eh_v.at[b],
                              sems.at[b]).wait()
        pltpu.make_async_copy(enc_hbm.at[pl.ds(0, CH)], rr_v.at[b],
                              sems.at[b]).wait()
        pltpu.make_async_copy(enc_hbm.at[pl.ds(0, CH)], et_v.at[b],
                              sems.at[b]).wait()

    def compute(c, b):
        for g in range(CH // L):
            def row_body(j, out16):
                row = g * L + j
                acc = [jnp.zeros((L,), jnp.float32) for _ in range(4)]
                for i in range(NVEC2):
                    sl = pl.ds(i * L, L)
                    p = (plsc.bitcast(eh_v[b, row, sl], jnp.bfloat16)
                         * plsc.bitcast(rr_v[b, row, sl], jnp.bfloat16)
                         * plsc.bitcast(et_v[b, row, sl], jnp.bfloat16))
                    pa, pb = plsc.unpack(p,
                                         format=plsc.PackFormat.INTERLEAVED)
                    k = 2 * (i % 2)
                    acc[k] = acc[k] + pa
                    acc[k + 1] = acc[k + 1] + pb
                s = jnp.sum((acc[0] + acc[1]) + (acc[2] + acc[3]))
                return jnp.where(lane == j, s, out16)

            out16 = lax.fori_loop(0, L, row_body,
                                  jnp.zeros((L,), jnp.float32), unroll=False)
            outv_v[pl.ds(g * L, L)] = out16

        pltpu.sync_copy(outv_v, out_hbm.at[pl.ds(chunk_start(c), CH)])

    fetch(0, 0)

    def outer(i, _):
        for b in (0, 1):
            c = i * 2 + b

            @pl.when(c < NCHUNK - 1)
            def _fire():
                fetch(c + 1, 1 - b)

            drain(b)
            compute(c, b)
        return _

    lax.fori_loop(0, NCHUNK // 2, outer, None, unroll=False)


@jax.jit
def _dist_mult_sc(enc, h, r, t, rel_weight):
    mesh = plsc.VectorSubcoreMesh(core_axis_name="c", subcore_axis_name="s",
                                  num_cores=NC, num_subcores=NS)
    return pl.kernel(
        _body,
        out_type=jax.ShapeDtypeStruct((E,), jnp.float32),
        mesh=mesh,
        compiler_params=pltpu.CompilerParams(needs_layout_passes=False),
        scratch_types=[
            pltpu.VMEM((2 * CH,), jnp.int32),
            pltpu.VMEM((2 * CH,), jnp.int32),
            pltpu.VMEM((2 * CH,), jnp.int32),
            pltpu.VMEM((2, CH, D // 2), jnp.int32),
            pltpu.VMEM((2, CH, D // 2), jnp.int32),
            pltpu.VMEM((2, CH, D // 2), jnp.int32),
            pltpu.VMEM((CH,), jnp.float32),
            pltpu.SemaphoreType.DMA((2,)),
        ],
    )(enc, h, r, t, rel_weight)


def _pack_i32(table):
    tb = table.astype(jnp.bfloat16)
    return lax.bitcast_convert_type(tb.reshape(table.shape[0], -1, 2),
                                    jnp.int32)


def kernel(enc, h, r, t, rel_weight):
    return _dist_mult_sc(_pack_i32(enc),
                         jnp.asarray(h, jnp.int32),
                         jnp.asarray(r, jnp.int32),
                         jnp.asarray(t, jnp.int32),
                         _pack_i32(rel_weight))


# speedup vs baseline: 1.3122x; 1.1010x over previous
import jax
import jax.numpy as jnp
from jax import lax
from jax.experimental import pallas as pl
from jax.experimental.pallas import tpu as pltpu
from jax.experimental.pallas import tpu_sc as plsc

N, D = 10000, 256
E = 160000
NUM_REL = 500

NC, NS, L = 2, 16, 16
NW = NC * NS
PW = E // NW
CH = 160
LAST_START = PW - CH
NCHUNK = 32
NVEC2 = D // (2 * L)


def _body(enc_hbm, h_hbm, r_hbm, t_hbm, rel_hbm, out_hbm,
          idxh_v, idxr_v, idxt_v, eh_v, rr_v, et_v, outv_v, sems, isems):
    wid = lax.axis_index("s") * NC + lax.axis_index("c")
    base = wid * PW
    lane = lax.iota(jnp.int32, L)

    def chunk_start(c):
        return base + jnp.minimum(c * CH, LAST_START)

    def fire_idx(c, p):
        start = chunk_start(c)
        ib = pl.ds(p * CH, CH)
        pltpu.async_copy(h_hbm.at[pl.ds(start, CH)], idxh_v.at[ib],
                         isems.at[p])
        pltpu.async_copy(r_hbm.at[pl.ds(start, CH)], idxr_v.at[ib],
                         isems.at[p])
        pltpu.async_copy(t_hbm.at[pl.ds(start, CH)], idxt_v.at[ib],
                         isems.at[p])

    def wait_idx(p):
        ib = pl.ds(p * CH, CH)
        pltpu.make_async_copy(h_hbm.at[pl.ds(0, CH)], idxh_v.at[ib],
                              isems.at[p]).wait()
        pltpu.make_async_copy(h_hbm.at[pl.ds(0, CH)], idxr_v.at[ib],
                              isems.at[p]).wait()
        pltpu.make_async_copy(h_hbm.at[pl.ds(0, CH)], idxt_v.at[ib],
                              isems.at[p]).wait()

    def fire_gather(b):
        ib = pl.ds(b * CH, CH)
        pltpu.async_copy(enc_hbm.at[idxh_v.at[ib]], eh_v.at[b], sems.at[b])
        pltpu.async_copy(rel_hbm.at[idxr_v.at[ib]], rr_v.at[b], sems.at[b])
        pltpu.async_copy(enc_hbm.at[idxt_v.at[ib]], et_v.at[b], sems.at[b])

    def drain(b):
        pltpu.make_async_copy(enc_hbm.at[pl.ds(0, CH)], eh_v.at[b],
                              sems.at[b]).wait()
        pltpu.make_async_copy(enc_hbm.at[pl.ds(0, CH)], rr_v.at[b],
                              sems.at[b]).wait()
        pltpu.make_async_copy(enc_hbm.at[pl.ds(0, CH)], et_v.at[b],
                              sems.at[b]).wait()

    def compute(c, b):
        for g in range(CH // L):
            def row_body(j, out16):
                row = g * L + j
                acc = [jnp.zeros((L,), jnp.float32) for _ in range(4)]
                for i in range(NVEC2):
                    sl = pl.ds(i * L, L)
                    p = (plsc.bitcast(eh_v[b, row, sl], jnp.bfloat16)
                         * plsc.bitcast(rr_v[b, row, sl], jnp.bfloat16)
                         * plsc.bitcast(et_v[b, row, sl], jnp.bfloat16))
                    pa, pb = plsc.unpack(p,
                                         format=plsc.PackFormat.INTERLEAVED)
                    k = 2 * (i % 2)
                    acc[k] = acc[k] + pa
                    acc[k + 1] = acc[k + 1] + pb
                s = jnp.sum((acc[0] + acc[1]) + (acc[2] + acc[3]))
                return jnp.where(lane == j, s, out16)

            out16 = lax.fori_loop(0, L, row_body,
                                  jnp.zeros((L,), jnp.float32), unroll=False)
            outv_v[pl.ds(g * L, L)] = out16

        pltpu.sync_copy(outv_v, out_hbm.at[pl.ds(chunk_start(c), CH)])

    fire_idx(0, 0)
    fire_idx(1, 1)
    wait_idx(0)
    fire_gather(0)

    def outer(i, _):
        for b in (0, 1):
            c = i * 2 + b

            @pl.when(c < NCHUNK - 1)
            def _fire():
                wait_idx(1 - b)
                fire_gather(1 - b)

            drain(b)

            @pl.when(c < NCHUNK - 2)
            def _prefetch():
                fire_idx(c + 2, b)

            compute(c, b)
        return _

    lax.fori_loop(0, NCHUNK // 2, outer, None, unroll=False)


@jax.jit
def _dist_mult_sc(enc, h, r, t, rel_weight):
    mesh = plsc.VectorSubcoreMesh(core_axis_name="c", subcore_axis_name="s",
                                  num_cores=NC, num_subcores=NS)
    return pl.kernel(
        _body,
        out_type=jax.ShapeDtypeStruct((E,), jnp.float32),
        mesh=mesh,
        compiler_params=pltpu.CompilerParams(needs_layout_passes=False),
        scratch_types=[
            pltpu.VMEM((2 * CH,), jnp.int32),
            pltpu.VMEM((2 * CH,), jnp.int32),
            pltpu.VMEM((2 * CH,), jnp.int32),
            pltpu.VMEM((2, CH, D // 2), jnp.int32),
            pltpu.VMEM((2, CH, D // 2), jnp.int32),
            pltpu.VMEM((2, CH, D // 2), jnp.int32),
            pltpu.VMEM((CH,), jnp.float32),
            pltpu.SemaphoreType.DMA((2,)),
            pltpu.SemaphoreType.DMA((2,)),
        ],
    )(enc, h, r, t, rel_weight)


def _pack_i32(table):
    tb = table.astype(jnp.bfloat16)
    return lax.bitcast_convert_type(tb.reshape(table.shape[0], -1, 2),
                                    jnp.int32)


def kernel(enc, h, r, t, rel_weight):
    return _dist_mult_sc(_pack_i32(enc),
                         jnp.asarray(h, jnp.int32),
                         jnp.asarray(r, jnp.int32),
                         jnp.asarray(t, jnp.int32),
                         _pack_i32(rel_weight))
